# baseline (speedup 1.0000x reference)
import functools

import jax
import jax.numpy as jnp
from jax.experimental import pallas as pl
from jax.experimental.pallas import tpu as pltpu

_EPS = 1e-3


def _head_kernel(x_ref, w1_ref, w2_ref, wc_ref, wo_ref,
                 b1_ref, g1_ref, be1_ref, m1_ref, v1_ref,
                 b2_ref, g2_ref, be2_ref, m2_ref, v2_ref,
                 bc_ref, bo_ref,
                 logit_ref, prob_ref, off_ref,
                 acc_ref, *, nh: int, bm: int):
    h = pl.program_id(0)
    i = pl.program_id(1)
    rows = pl.ds(i * bm, bm)

    kw = x_ref.shape[2] * x_ref.shape[3]
    xv = x_ref[:, 0, :, :].reshape(bm, kw).astype(jnp.bfloat16)
    wv = w1_ref[0].reshape(kw, w1_ref.shape[3]).astype(jnp.bfloat16)
    part = jnp.dot(xv, wv, preferred_element_type=jnp.float32)

    @pl.when(h == 0)
    def _():
        acc_ref[rows, :] = part

    @pl.when(h > 0)
    def _():
        acc_ref[rows, :] += part

    @pl.when(h == nh - 1)
    def _():
        s1 = g1_ref[...] * jax.lax.rsqrt(v1_ref[...] + _EPS)
        t1 = (b1_ref[...] - m1_ref[...]) * s1 + be1_ref[...]
        y1 = jnp.maximum(acc_ref[rows, :] * s1 + t1, 0.0).astype(jnp.bfloat16)

        z2 = jnp.dot(y1, w2_ref[0, 0].astype(jnp.bfloat16),
                     preferred_element_type=jnp.float32)
        s2 = g2_ref[...] * jax.lax.rsqrt(v2_ref[...] + _EPS)
        t2 = (b2_ref[...] - m2_ref[...]) * s2 + be2_ref[...]
        y2 = jnp.maximum(z2 * s2 + t2, 0.0).astype(jnp.bfloat16)

        logit = jnp.dot(y2, wc_ref[...].astype(jnp.bfloat16),
                        preferred_element_type=jnp.float32) + bc_ref[...]
        logit_ref[rows, :] = logit
        mx = jnp.max(logit, axis=-1, keepdims=True)
        e = jnp.exp(logit - mx)
        prob_ref[rows, :] = e / jnp.sum(e, axis=-1, keepdims=True)

        off_ref[rows, :] = jnp.dot(y2, wo_ref[...].astype(jnp.bfloat16),
                                   preferred_element_type=jnp.float32) + bo_ref[...]


def kernel(inputs, W1, b1, g1, be1, m1, v1, W2, b2, g2, be2, m2, v2, Wc, bc, Wo, bo):
    n, p, _, c = inputs.shape
    h = W1.shape[-1]
    ncls = Wc.shape[-1]
    no = Wo.shape[-1]

    bm = 200 if n % 200 == 0 else n
    nm = n // bm

    row = lambda a: a.reshape(1, -1)
    vec_spec = pl.BlockSpec((1, h), lambda k, i: (0, 0))

    out = pl.pallas_call(
        functools.partial(_head_kernel, nh=p, bm=bm),
        grid=(p, nm),
        in_specs=[
            pl.BlockSpec((bm, 1, p, c), lambda k, i: (i, k, 0, 0)),
            pl.BlockSpec((1, p, c, h), lambda k, i: (k, 0, 0, 0)),
            pl.BlockSpec((1, 1, h, h), lambda k, i: (0, 0, 0, 0)),
            pl.BlockSpec((h, ncls), lambda k, i: (0, 0)),
            pl.BlockSpec((h, no), lambda k, i: (0, 0)),
            vec_spec, vec_spec, vec_spec, vec_spec, vec_spec,
            vec_spec, vec_spec, vec_spec, vec_spec, vec_spec,
            pl.BlockSpec((1, ncls), lambda k, i: (0, 0)),
            pl.BlockSpec((1, no), lambda k, i: (0, 0)),
        ],
        out_specs=[
            pl.BlockSpec((n, ncls), lambda k, i: (0, 0)),
            pl.BlockSpec((n, ncls), lambda k, i: (0, 0)),
            pl.BlockSpec((n, no), lambda k, i: (0, 0)),
        ],
        out_shape=[
            jax.ShapeDtypeStruct((n, ncls), jnp.float32),
            jax.ShapeDtypeStruct((n, ncls), jnp.float32),
            jax.ShapeDtypeStruct((n, no), jnp.float32),
        ],
        scratch_shapes=[pltpu.VMEM((n, h), jnp.float32)],
        compiler_params=pltpu.CompilerParams(
            dimension_semantics=("arbitrary", "arbitrary"),
        ),
    )(inputs, W1, W2, Wc, Wo,
      row(b1), row(g1), row(be1), row(m1), row(v1),
      row(b2), row(g2), row(be2), row(m2), row(v2),
      row(bc), row(bo))

    class_logit, class_prob, off = out
    return (class_logit, class_prob, off.reshape(n, ncls, 4))


# speedup vs baseline: 1.1211x; 1.1211x over previous
import functools

import jax
import jax.numpy as jnp
from jax.experimental import pallas as pl
from jax.experimental.pallas import tpu as pltpu

_EPS = 1e-3


def _head_kernel(x_ref, w1_ref, w2_ref, wc_ref, wo_ref,
                 b1_ref, g1_ref, be1_ref, m1_ref, v1_ref,
                 b2_ref, g2_ref, be2_ref, m2_ref, v2_ref,
                 bc_ref, bo_ref,
                 logit_ref, prob_ref, off_ref,
                 acc_ref, *, nh: int, bm: int):
    i = pl.program_id(0)
    h = pl.program_id(1)

    kw = x_ref.shape[2] * x_ref.shape[3]
    xv = x_ref[:, 0, :, :].reshape(bm, kw).astype(jnp.bfloat16)
    wv = w1_ref[0].reshape(kw, w1_ref.shape[3])
    part = jnp.dot(xv, wv, preferred_element_type=jnp.float32)

    @pl.when(h == 0)
    def _():
        acc_ref[...] = part

    @pl.when(h > 0)
    def _():
        acc_ref[...] += part

    @pl.when(h == nh - 1)
    def _():
        s1 = g1_ref[...] * jax.lax.rsqrt(v1_ref[...] + _EPS)
        t1 = (b1_ref[...] - m1_ref[...]) * s1 + be1_ref[...]
        y1 = jnp.maximum(acc_ref[...] * s1 + t1, 0.0).astype(jnp.bfloat16)

        z2 = jnp.dot(y1, w2_ref[...],
                     preferred_element_type=jnp.float32)
        s2 = g2_ref[...] * jax.lax.rsqrt(v2_ref[...] + _EPS)
        t2 = (b2_ref[...] - m2_ref[...]) * s2 + be2_ref[...]
        y2 = jnp.maximum(z2 * s2 + t2, 0.0).astype(jnp.bfloat16)

        logit = jnp.dot(y2, wc_ref[...].astype(jnp.bfloat16),
                        preferred_element_type=jnp.float32) + bc_ref[...]
        logit_ref[...] = logit
        mx = jnp.max(logit, axis=-1, keepdims=True)
        e = jnp.exp(logit - mx)
        prob_ref[...] = e / jnp.sum(e, axis=-1, keepdims=True)

        off_ref[...] = jnp.dot(y2, wo_ref[...].astype(jnp.bfloat16),
                                   preferred_element_type=jnp.float32) + bo_ref[...]


def kernel(inputs, W1, b1, g1, be1, m1, v1, W2, b2, g2, be2, m2, v2, Wc, bc, Wo, bo):
    n, p, _, c = inputs.shape
    h = W1.shape[-1]
    ncls = Wc.shape[-1]
    no = Wo.shape[-1]

    bm = 1000 if n % 1000 == 0 else n
    nm = n // bm

    w1bf = W1.astype(jnp.bfloat16)
    w2bf = W2.reshape(h, h).astype(jnp.bfloat16)

    row = lambda a: a.reshape(1, -1)
    vec_spec = pl.BlockSpec((1, h), lambda k, i: (0, 0))

    out = pl.pallas_call(
        functools.partial(_head_kernel, nh=p, bm=bm),
        grid=(nm, p),
        in_specs=[
            pl.BlockSpec((bm, 1, p, c), lambda i, k: (i, k, 0, 0)),
            pl.BlockSpec((1, p, c, h), lambda i, k: (k, 0, 0, 0)),
            pl.BlockSpec((h, h), lambda i, k: (0, 0)),
            pl.BlockSpec((h, ncls), lambda k, i: (0, 0)),
            pl.BlockSpec((h, no), lambda k, i: (0, 0)),
            vec_spec, vec_spec, vec_spec, vec_spec, vec_spec,
            vec_spec, vec_spec, vec_spec, vec_spec, vec_spec,
            pl.BlockSpec((1, ncls), lambda k, i: (0, 0)),
            pl.BlockSpec((1, no), lambda k, i: (0, 0)),
        ],
        out_specs=[
            pl.BlockSpec((bm, ncls), lambda i, k: (i, 0)),
            pl.BlockSpec((bm, ncls), lambda i, k: (i, 0)),
            pl.BlockSpec((bm, no), lambda i, k: (i, 0)),
        ],
        out_shape=[
            jax.ShapeDtypeStruct((n, ncls), jnp.float32),
            jax.ShapeDtypeStruct((n, ncls), jnp.float32),
            jax.ShapeDtypeStruct((n, no), jnp.float32),
        ],
        scratch_shapes=[pltpu.VMEM((bm, h), jnp.float32)],
        compiler_params=pltpu.CompilerParams(
            dimension_semantics=("arbitrary", "arbitrary"),
        ),
    )(inputs, w1bf, w2bf, Wc, Wo,
      row(b1), row(g1), row(be1), row(m1), row(v1),
      row(b2), row(g2), row(be2), row(m2), row(v2),
      row(bc), row(bo))

    class_logit, class_prob, off = out
    return (class_logit, class_prob, off.reshape(n, ncls, 4))
